# baseline (speedup 1.0000x reference)
import functools
import jax
import jax.numpy as jnp
from jax import lax
from jax.experimental import pallas as pl
from jax.experimental.pallas import tpu as pltpu
from jax.experimental.pallas import tpu_sc as plsc

B, T, N, D = 8, 50, 1024, 128
ROWS = B * T * N
NW = 32
ROWS_PER_W = ROWS // NW
R = 128
C = ROWS_PER_W // R
L = 16


def _sc_body(x_hbm, emb_hbm, out_hbm, emb_v, in0, in1, ou0, ou1,
             sin0, sin1, sout0, sout1):
    cid = lax.axis_index("c")
    sid = lax.axis_index("s")
    wid = sid * 2 + cid
    base = wid * ROWS_PER_W

    pltpu.sync_copy(emb_hbm, emb_v)

    ins = (in0, in1)
    ous = (ou0, ou1)
    sins = (sin0, sin1)
    souts = (sout0, sout1)

    def start_in(c, b):
        pltpu.async_copy(x_hbm.at[pl.ds(base + c * R, R), :], ins[b], sins[b])

    def wait_in(b):
        pltpu.make_async_copy(x_hbm.at[pl.ds(0, R), :], ins[b], sins[b]).wait()

    def start_out(c, b):
        pltpu.async_copy(ous[b], out_hbm.at[pl.ds(base + c * R, R), :], souts[b])

    def wait_out(b):
        pltpu.make_async_copy(ous[b], out_hbm.at[pl.ds(0, R), :], souts[b]).wait()

    start_in(0, 0)

    def group(g, first):
        for b in range(2):
            c = 2 * g + b
            nb = 1 - b
            if first and b == 0:
                start_in(jnp.int32(1), nb)
            else:
                @pl.when(c + 1 < C)
                def _():
                    start_in(c + 1, nb)
            wait_in(b)
            if not first:
                wait_out(b)
            t = ((base + c * R) // N) % T
            regs = [emb_v[t, pl.ds(L * v, L)] for v in range(D // L)]

            def row(r, _):
                for v in range(D // L):
                    ous[b][r, pl.ds(L * v, L)] = ins[b][r, pl.ds(L * v, L)] + regs[v]
                return 0

            lax.fori_loop(0, R, row, 0, unroll=2)
            start_out(c, b)

    group(jnp.int32(0), True)
    lax.fori_loop(1, C // 2, lambda g, _: (group(g, False), 0)[1], 0)
    wait_out(0)
    wait_out(1)


def kernel(x, emb_table):
    xf = x.reshape(ROWS, D)
    mesh = plsc.VectorSubcoreMesh(core_axis_name="c", subcore_axis_name="s")
    out = pl.kernel(
        _sc_body,
        out_type=jax.ShapeDtypeStruct((ROWS, D), jnp.float32),
        mesh=mesh,
        scratch_types=[
            pltpu.VMEM((T, D), jnp.float32),
            pltpu.VMEM((R, D), jnp.float32),
            pltpu.VMEM((R, D), jnp.float32),
            pltpu.VMEM((R, D), jnp.float32),
            pltpu.VMEM((R, D), jnp.float32),
            pltpu.SemaphoreType.DMA,
            pltpu.SemaphoreType.DMA,
            pltpu.SemaphoreType.DMA,
            pltpu.SemaphoreType.DMA,
        ],
    )(xf, emb_table)
    return out.reshape(B, T, N, D)


if __name__ == "__main__":
    import numpy as np
    d_x = jax.random.normal(jax.random.key(1), (B, T, N, D), dtype=jnp.float32)
    d_e = jax.random.normal(jax.random.key(2), (T, D), dtype=jnp.float32)
    out = sc_kernel(d_x, d_e)
    ref = d_x + d_e.reshape(1, T, 1, D)
    print("max abs err", float(jnp.max(jnp.abs(out - ref))))


# speedup vs baseline: 2.9330x; 2.9330x over previous
import jax
import jax.numpy as jnp
from jax import lax
from jax.experimental import pallas as pl
from jax.experimental.pallas import tpu as pltpu
from jax.experimental.pallas import tpu_sc as plsc

B, T, N, D = 8, 50, 1024, 128
ROWS = B * T * N
NW = 32
ROWS_PER_W = ROWS // NW
R = 128
C = ROWS_PER_W // R
NB = 4
PF = 2
L = 16
GROUPS = C // NB


def _sc_body(x_hbm, emb_hbm, out_hbm, emb_v, b0, b1, b2, b3,
             si0, si1, si2, si3, so0, so1, so2, so3):
    cid = lax.axis_index("c")
    sid = lax.axis_index("s")
    wid = sid * 2 + cid
    base = wid * ROWS_PER_W

    pltpu.sync_copy(emb_hbm, emb_v)

    bufs = (b0, b1, b2, b3)
    sins = (si0, si1, si2, si3)
    souts = (so0, so1, so2, so3)

    def start_in(c, b):
        pltpu.async_copy(x_hbm.at[pl.ds(base + c * R, R), :], bufs[b], sins[b])

    def wait_in(b):
        pltpu.make_async_copy(x_hbm.at[pl.ds(0, R), :], bufs[b], sins[b]).wait()

    def start_out(c, b):
        pltpu.async_copy(bufs[b], out_hbm.at[pl.ds(base + c * R, R), :], souts[b])

    def wait_out(b):
        pltpu.make_async_copy(bufs[b], out_hbm.at[pl.ds(0, R), :], souts[b]).wait()

    def compute(c, b):
        t = ((base + c * R) // N) % T
        regs = [emb_v[t, pl.ds(L * v, L)] for v in range(D // L)]

        def row(r, _):
            for v in range(D // L):
                plsc.addupdate(bufs[b].at[r, pl.ds(L * v, L)], regs[v])
            return 0

        lax.fori_loop(0, R, row, 0, unroll=2)

    def step(c, b, *, guard_out, guard_pf):
        p = c + PF
        pb = (b + PF) % NB
        if guard_out:
            wait_out(pb)
        if guard_pf:
            @pl.when(p < C)
            def _():
                start_in(p, pb)
        else:
            start_in(p, pb)
        wait_in(b)
        compute(c, b)
        start_out(c, b)

    for b in range(PF):
        start_in(jnp.int32(b), b)
    for b in range(NB):
        step(jnp.int32(b), b, guard_out=(b + PF >= NB), guard_pf=False)

    def group(g, _):
        for b in range(NB):
            step(g * NB + b, b, guard_out=True, guard_pf=True)
        return 0

    lax.fori_loop(1, GROUPS, group, 0)

    for c in range(C - PF, C):
        wait_out(c % NB)


def kernel(x, emb_table):
    xf = x.reshape(ROWS, D)
    mesh = plsc.VectorSubcoreMesh(core_axis_name="c", subcore_axis_name="s")
    out = pl.kernel(
        _sc_body,
        out_type=jax.ShapeDtypeStruct((ROWS, D), jnp.float32),
        mesh=mesh,
        scratch_types=[
            pltpu.VMEM((T, D), jnp.float32),
            *[pltpu.VMEM((R, D), jnp.float32) for _ in range(NB)],
            *[pltpu.SemaphoreType.DMA for _ in range(2 * NB)],
        ],
    )(xf, emb_table)
    return out.reshape(B, T, N, D)
